# baseline (speedup 1.0000x reference)
import functools

import jax
import jax.numpy as jnp
from jax import lax
from jax.experimental import pallas as pl
from jax.experimental.pallas import tpu as pltpu
from jax.experimental.pallas import tpu_sc as plsc

N_NODES = 10000
E_EDGES = 160000
LANE = 16
NC, NS = 2, 16
NW = NC * NS
ROWS_PAD = 10240
CH = 2000
GB = 128
CHP = 2048


def _seg_reduce_body(n_tables, is_max, n_rng, rows_per, nchunk,
                     x_hbm, src_hbm, dst_hbm, w_hbm, out_hbm,
                     acc, srcb, dstb, wb, srcc, ldstc, wc, stage, sem):
    wid = lax.axis_index("c") * NS + lax.axis_index("s")
    rid = wid % n_rng
    tid = wid // n_rng
    lo = rid * rows_per

    init_val = -jnp.inf if is_max else 0.0

    def init_body(i, _):
        r = i // 8
        j = i % 8
        acc[r, pl.ds(j * LANE, LANE)] = jnp.full((LANE,), init_val, jnp.float32)
        return 0

    lax.fori_loop(0, rows_per * 8, init_body, 0)

    def zero_idx(i, _):
        srcc[pl.ds(i * LANE, LANE)] = jnp.zeros((LANE,), jnp.int32)
        return 0

    lax.fori_loop(0, CHP // LANE, zero_idx, 0)

    def chunk_body(ci, _):
        e0 = ci * CH
        pltpu.sync_copy(src_hbm.at[pl.ds(e0, CH)], srcb)
        pltpu.sync_copy(dst_hbm.at[pl.ds(e0, CH)], dstb)
        pltpu.sync_copy(w_hbm.at[pl.ds(e0, CH)], wb)

        def scan_body(i, mv):
            dv = dstb[pl.ds(i * LANE, LANE)]
            sv = srcb[pl.ds(i * LANE, LANE)]
            wv = wb[pl.ds(i * LANE, LANE)]
            ldv = dv - lo
            msk = (ldv >= 0) & (ldv < rows_per)
            inc = lax.cumsum(msk.astype(jnp.int32))
            pos = jnp.maximum(mv + inc - 1, 0)
            plsc.store_scatter(srcc, [pos], sv + tid * N_NODES, mask=msk)
            plsc.store_scatter(ldstc, [pos], ldv, mask=msk)
            plsc.store_scatter(wc, [pos], wv, mask=msk)
            return mv + plsc.all_reduce_population_count(msk)

        mv = lax.fori_loop(0, CH // LANE, scan_body,
                           jnp.zeros((LANE,), jnp.int32))
        m = mv[0]

        pad_iota = m + lax.iota(jnp.int32, LANE)
        plsc.store_scatter(srcc, [pad_iota], jnp.zeros((LANE,), jnp.int32))
        plsc.store_scatter(ldstc, [pad_iota],
                           jnp.full((LANE,), rows_per, jnp.int32))
        plsc.store_scatter(wc, [pad_iota], jnp.zeros((LANE,), jnp.float32))
        m_pad = ((m + (LANE - 1)) // LANE) * LANE

        nb = (m_pad + (GB - 1)) // GB

        def batch_body(b, _):
            g0 = b * GB
            cp = pltpu.async_copy(x_hbm.at[srcc.at[pl.ds(g0, GB)]], stage, sem)
            cp.wait()
            n_grp = jnp.minimum(m_pad - g0, GB) // LANE

            def grp_body(gi, _):
                eb = gi * LANE
                ld16 = ldstc[pl.ds(g0 + eb, LANE)]
                w16 = wc[pl.ds(g0 + eb, LANE)]
                for lane in range(LANE):
                    ld = ld16[lane]
                    ws = w16[lane]
                    e = eb + lane
                    for j in range(8):
                        g = stage[e, pl.ds(j * LANE, LANE)]
                        msg = g * ws
                        a = acc[ld, pl.ds(j * LANE, LANE)]
                        if is_max:
                            acc[ld, pl.ds(j * LANE, LANE)] = jnp.maximum(a, msg)
                        else:
                            acc[ld, pl.ds(j * LANE, LANE)] = a + msg
                return 0

            lax.fori_loop(0, n_grp, grp_body, 0)
            return 0

        lax.fori_loop(0, nb, batch_body, 0)
        return 0

    lax.fori_loop(0, nchunk, chunk_body, 0)

    if is_max:
        def fix_body(i, _):
            r = i // 8
            j = i % 8
            v = acc[r, pl.ds(j * LANE, LANE)]
            acc[r, pl.ds(j * LANE, LANE)] = jnp.where(
                v == -jnp.inf, jnp.zeros((LANE,), jnp.float32), v)
            return 0

        lax.fori_loop(0, rows_per * 8, fix_body, 0)

    pltpu.sync_copy(acc.at[pl.ds(0, rows_per)],
                    out_hbm.at[pl.ds(tid * ROWS_PAD + lo, rows_per)])


def _seg_reduce(x_flat, src, dst, w, *, n_tables, is_max):
    e = src.shape[0]
    assert e % CH == 0
    n_rng = NW // n_tables
    rows_per = ROWS_PAD // n_rng
    mesh = plsc.VectorSubcoreMesh(core_axis_name="c", subcore_axis_name="s")
    body = functools.partial(_seg_reduce_body, n_tables, is_max, n_rng,
                             rows_per, e // CH)
    fn = pl.kernel(
        body,
        out_type=jax.ShapeDtypeStruct((n_tables * ROWS_PAD, 128), jnp.float32),
        mesh=mesh,
        scratch_types=[
            pltpu.VMEM((rows_per + 8, 128), jnp.float32),
            pltpu.VMEM((CH,), jnp.int32),
            pltpu.VMEM((CH,), jnp.int32),
            pltpu.VMEM((CH,), jnp.float32),
            pltpu.VMEM((CHP,), jnp.int32),
            pltpu.VMEM((CHP,), jnp.int32),
            pltpu.VMEM((CHP,), jnp.float32),
            pltpu.VMEM((GB, 128), jnp.float32),
            pltpu.SemaphoreType.DMA,
        ],
        name=("seg_max" if is_max else "seg_sum") + f"_{n_tables}",
        compiler_params=pltpu.CompilerParams(needs_layout_passes=False),
    )
    return fn(x_flat, src, dst, w)


_BR = 1000


def _mm_body(a_ref, w_ref, o_ref):
    o_ref[...] = jnp.dot(a_ref[...], w_ref[...],
                         preferred_element_type=jnp.float32)


def _mm(a, w):
    m, k = a.shape
    n = w.shape[1]
    assert m % _BR == 0
    return pl.pallas_call(
        _mm_body,
        grid=(m // _BR,),
        in_specs=[pl.BlockSpec((_BR, k), lambda i: (i, 0)),
                  pl.BlockSpec((k, n), lambda i: (0, 0))],
        out_specs=pl.BlockSpec((_BR, n), lambda i: (i, 0)),
        out_shape=jax.ShapeDtypeStruct((m, n), jnp.float32),
    )(a, w)


def _fused_body(c_ref, a_ref, w_ref, b_ref, o_ref):
    x = c_ref[...] + jnp.dot(a_ref[...], w_ref[...],
                             preferred_element_type=jnp.float32) + b_ref[...]
    o_ref[...] = jnp.where(x >= 0, x, 0.01 * x)


def _fused(c, a, w, b):
    m, k = a.shape
    n = w.shape[1]
    assert m % _BR == 0 and c.shape == (m, n)
    return pl.pallas_call(
        _fused_body,
        grid=(m // _BR,),
        in_specs=[pl.BlockSpec((_BR, n), lambda i: (i, 0)),
                  pl.BlockSpec((_BR, k), lambda i: (i, 0)),
                  pl.BlockSpec((k, n), lambda i: (0, 0)),
                  pl.BlockSpec((1, n), lambda i: (0, 0))],
        out_specs=pl.BlockSpec((_BR, n), lambda i: (i, 0)),
        out_shape=jax.ShapeDtypeStruct((m, n), jnp.float32),
    )(c, a, w, b.reshape(1, n))


def kernel(x_metric, x_alert, edge_index_corr, edge_index_cause,
           edge_weight_corr, edge_weight_cause,
           W_rel_corr_0, b_rel_corr_0, W_root_corr_0,
           W_rel_cause_0, b_rel_cause_0, W_root_cause_0,
           W_rel_corr_1, b_rel_corr_1, W_root_corr_1,
           W_rel_cause_1, b_rel_cause_1, W_root_cause_1):
    xm = x_metric
    xa = x_alert
    src_c = edge_index_corr[0].astype(jnp.int32)
    dst_c = edge_index_corr[1].astype(jnp.int32)
    src_a = edge_index_cause[0].astype(jnp.int32)
    dst_a = edge_index_cause[1].astype(jnp.int32)

    p = _mm(xm, W_rel_corr_0)
    r_a = _mm(xa, W_root_cause_0)

    s = _seg_reduce(p, src_c, dst_c, edge_weight_corr,
                    n_tables=1, is_max=False)[:N_NODES]
    xm_blk = jnp.concatenate([xm[:, :128], xm[:, 128:]], axis=0)
    m0f = _seg_reduce(xm_blk, src_a, dst_a, edge_weight_cause,
                      n_tables=2, is_max=True)
    m0 = jnp.concatenate([m0f[:N_NODES], m0f[ROWS_PAD:ROWS_PAD + N_NODES]],
                         axis=1)

    xm1 = _fused(s, xm, W_root_corr_0, b_rel_corr_0)
    xa1 = _fused(r_a, m0, W_rel_cause_0, b_rel_cause_0)

    m1 = _seg_reduce(xm1, src_a, dst_a, edge_weight_cause,
                     n_tables=1, is_max=True)[:N_NODES]
    t = _mm(m1, W_rel_cause_1)
    xa2 = _fused(t, xa1, W_root_cause_1, b_rel_cause_1)
    return xa2


# speedup vs baseline: 1.0016x; 1.0016x over previous
import functools

import jax
import jax.numpy as jnp
from jax import lax
from jax.experimental import pallas as pl
from jax.experimental.pallas import tpu as pltpu
from jax.experimental.pallas import tpu_sc as plsc

N_NODES = 10000
E_EDGES = 160000
LANE = 16
NC, NS = 2, 16
NW = NC * NS
ROWS_PAD = 10240
CH = 1600
NCHUNK = E_EDGES // CH
GB = 96
CHP = 1632


def _seg_reduce_body(is_max, n_rng, rows_per, x_hbm, edges_hbm, out_hbm,
                     acc, eb0, eb1, srcc, ldstc, wc, st0, st1,
                     esem0, esem1, gsem0, gsem1):
    wid = lax.axis_index("c") * NS + lax.axis_index("s")
    rid = wid % n_rng
    tid = wid // n_rng
    lo = rid * rows_per
    accn = rows_per * 128

    init_val = -jnp.inf if is_max else 0.0
    iota = lax.iota(jnp.int32, LANE)

    def init_body(i, _):
        acc[pl.ds(i * LANE, LANE)] = jnp.full((LANE,), init_val, jnp.float32)
        return 0

    lax.fori_loop(0, (accn + 128) // LANE, init_body, 0)

    def zero_idx(i, _):
        srcc[pl.ds(i * LANE, LANE)] = jnp.zeros((LANE,), jnp.int32)
        ldstc[pl.ds(i * LANE, LANE)] = jnp.zeros((LANE,), jnp.int32)
        wc[pl.ds(i * LANE, LANE)] = jnp.zeros((LANE,), jnp.float32)
        return 0

    lax.fori_loop(0, CHP // LANE, zero_idx, 0)

    def issue_chunk(ebuf, esem, ci):
        pltpu.async_copy(edges_hbm.at[pl.ds(ci * 3 * CH, 3 * CH)], ebuf, esem)

    def wait_chunk(ebuf, esem, ci):
        pltpu.make_async_copy(edges_hbm.at[pl.ds(ci * 3 * CH, 3 * CH)],
                              ebuf, esem).wait()

    def issue_gather(stage, gsem, g0):
        pltpu.async_copy(x_hbm.at[srcc.at[pl.ds(g0, GB)]], stage, gsem)

    def wait_gather(stage, gsem, g0):
        pltpu.make_async_copy(x_hbm.at[srcc.at[pl.ds(g0, GB)]],
                              stage, gsem).wait()

    def process_batch(stage, g0, m_pad):
        n_grp = jnp.minimum(m_pad - g0, GB) // LANE

        def grp_body(gi, _):
            eb = gi * LANE
            ld16 = ldstc[pl.ds(g0 + eb, LANE)]
            w16 = wc[pl.ds(g0 + eb, LANE)]
            rowb = ld16 * 128
            for lane in range(LANE):
                base = jnp.broadcast_to(rowb[lane], (LANE,))
                wbv = jnp.broadcast_to(w16[lane], (LANE,))
                for j in range(8):
                    addr = base + (j * LANE) + iota
                    g = stage[eb + lane, pl.ds(j * LANE, LANE)]
                    a = plsc.load_gather(acc, [addr])
                    msg = g * wbv
                    upd = jnp.maximum(a, msg) if is_max else a + msg
                    plsc.store_scatter(acc, [addr], upd)
            return 0

        lax.fori_loop(0, n_grp, grp_body, 0)

    def process_chunk(ebuf, _ci):
        def scan_body(i, mv):
            sv = ebuf[pl.ds(i * LANE, LANE)]
            dv = ebuf[pl.ds(CH + i * LANE, LANE)]
            wv = plsc.bitcast(ebuf[pl.ds(2 * CH + i * LANE, LANE)],
                              jnp.float32)
            ldv = dv - lo
            msk = (ldv >= 0) & (ldv < rows_per)
            inc = lax.cumsum(msk.astype(jnp.int32))
            pos = jnp.maximum(mv + inc - 1, 0)
            plsc.store_scatter(srcc, [pos], sv + tid * N_NODES, mask=msk)
            plsc.store_scatter(ldstc, [pos], ldv, mask=msk)
            plsc.store_scatter(wc, [pos], wv, mask=msk)
            return mv + plsc.all_reduce_population_count(msk)

        mv = lax.fori_loop(0, CH // LANE, scan_body,
                           jnp.zeros((LANE,), jnp.int32))
        plsc.store_scatter(srcc, [mv + iota], jnp.zeros((LANE,), jnp.int32))
        plsc.store_scatter(ldstc, [mv + iota],
                           jnp.full((LANE,), rows_per, jnp.int32))
        plsc.store_scatter(wc, [mv + iota], jnp.zeros((LANE,), jnp.float32))
        m = mv[0]
        m_pad = ((m + (LANE - 1)) // LANE) * LANE
        nb = (m_pad + (GB - 1)) // GB

        @pl.when(nb > 0)
        def _():
            issue_gather(st0, gsem0, 0)

            def batch_pair(i, _):
                b0 = 2 * i
                g0 = b0 * GB
                wait_gather(st0, gsem0, g0)

                @pl.when(b0 + 1 < nb)
                def _():
                    issue_gather(st1, gsem1, g0 + GB)

                process_batch(st0, g0, m_pad)

                @pl.when(b0 + 1 < nb)
                def _():
                    wait_gather(st1, gsem1, g0 + GB)

                    @pl.when(b0 + 2 < nb)
                    def _():
                        issue_gather(st0, gsem0, g0 + 2 * GB)

                    process_batch(st1, g0 + GB, m_pad)

                return 0

            lax.fori_loop(0, (nb + 1) // 2, batch_pair, 0)

    issue_chunk(eb0, esem0, 0)

    def chunk_pair(i, _):
        c0 = 2 * i
        wait_chunk(eb0, esem0, c0)
        issue_chunk(eb1, esem1, c0 + 1)
        process_chunk(eb0, c0)
        wait_chunk(eb1, esem1, c0 + 1)

        @pl.when(c0 + 2 < NCHUNK)
        def _():
            issue_chunk(eb0, esem0, c0 + 2)

        process_chunk(eb1, c0 + 1)
        return 0

    lax.fori_loop(0, NCHUNK // 2, chunk_pair, 0)

    if is_max:
        def fix_body(i, _):
            v = acc[pl.ds(i * LANE, LANE)]
            acc[pl.ds(i * LANE, LANE)] = jnp.where(
                v == -jnp.inf, jnp.zeros((LANE,), jnp.float32), v)
            return 0

        lax.fori_loop(0, accn // LANE, fix_body, 0)

    pltpu.sync_copy(acc.at[pl.ds(0, accn)],
                    out_hbm.at[pl.ds((tid * ROWS_PAD + lo) * 128, accn)])


def _seg_reduce(x_flat, edges_packed, *, n_tables, is_max):
    n_rng = NW // n_tables
    rows_per = ROWS_PAD // n_rng
    mesh = plsc.VectorSubcoreMesh(core_axis_name="c", subcore_axis_name="s")
    body = functools.partial(_seg_reduce_body, is_max, n_rng, rows_per)
    fn = pl.kernel(
        body,
        out_type=jax.ShapeDtypeStruct((n_tables * ROWS_PAD * 128,),
                                      jnp.float32),
        mesh=mesh,
        scratch_types=[
            pltpu.VMEM(((rows_per + 1) * 128,), jnp.float32),
            pltpu.VMEM((3 * CH,), jnp.int32),
            pltpu.VMEM((3 * CH,), jnp.int32),
            pltpu.VMEM((CHP,), jnp.int32),
            pltpu.VMEM((CHP,), jnp.int32),
            pltpu.VMEM((CHP,), jnp.float32),
            pltpu.VMEM((GB, 128), jnp.float32),
            pltpu.VMEM((GB, 128), jnp.float32),
            pltpu.SemaphoreType.DMA,
            pltpu.SemaphoreType.DMA,
            pltpu.SemaphoreType.DMA,
            pltpu.SemaphoreType.DMA,
        ],
        name=("seg_max" if is_max else "seg_sum") + f"_{n_tables}",
        compiler_params=pltpu.CompilerParams(needs_layout_passes=False),
    )
    return fn(x_flat, edges_packed).reshape(n_tables * ROWS_PAD, 128)


def _pack_edges(src, dst, w):
    s2 = src.reshape(NCHUNK, CH)
    d2 = dst.reshape(NCHUNK, CH)
    w2 = lax.bitcast_convert_type(w, jnp.int32).reshape(NCHUNK, CH)
    return jnp.concatenate([s2, d2, w2], axis=1).reshape(-1)


_BR = 1000


def _mm_body(a_ref, w_ref, o_ref):
    o_ref[...] = jnp.dot(a_ref[...], w_ref[...],
                         preferred_element_type=jnp.float32)


def _mm(a, w):
    m, k = a.shape
    n = w.shape[1]
    assert m % _BR == 0
    return pl.pallas_call(
        _mm_body,
        grid=(m // _BR,),
        in_specs=[pl.BlockSpec((_BR, k), lambda i: (i, 0)),
                  pl.BlockSpec((k, n), lambda i: (0, 0))],
        out_specs=pl.BlockSpec((_BR, n), lambda i: (i, 0)),
        out_shape=jax.ShapeDtypeStruct((m, n), jnp.float32),
    )(a, w)


def _fused_body(c_ref, a_ref, w_ref, b_ref, o_ref):
    x = c_ref[...] + jnp.dot(a_ref[...], w_ref[...],
                             preferred_element_type=jnp.float32) + b_ref[...]
    o_ref[...] = jnp.where(x >= 0, x, 0.01 * x)


def _fused(c, a, w, b):
    m, k = a.shape
    n = w.shape[1]
    assert m % _BR == 0 and c.shape == (m, n)
    return pl.pallas_call(
        _fused_body,
        grid=(m // _BR,),
        in_specs=[pl.BlockSpec((_BR, n), lambda i: (i, 0)),
                  pl.BlockSpec((_BR, k), lambda i: (i, 0)),
                  pl.BlockSpec((k, n), lambda i: (0, 0)),
                  pl.BlockSpec((1, n), lambda i: (0, 0))],
        out_specs=pl.BlockSpec((_BR, n), lambda i: (i, 0)),
        out_shape=jax.ShapeDtypeStruct((m, n), jnp.float32),
    )(c, a, w, b.reshape(1, n))


def kernel(x_metric, x_alert, edge_index_corr, edge_index_cause,
           edge_weight_corr, edge_weight_cause,
           W_rel_corr_0, b_rel_corr_0, W_root_corr_0,
           W_rel_cause_0, b_rel_cause_0, W_root_cause_0,
           W_rel_corr_1, b_rel_corr_1, W_root_corr_1,
           W_rel_cause_1, b_rel_cause_1, W_root_cause_1):
    xm = x_metric
    xa = x_alert
    ec = _pack_edges(edge_index_corr[0].astype(jnp.int32),
                     edge_index_corr[1].astype(jnp.int32), edge_weight_corr)
    ea = _pack_edges(edge_index_cause[0].astype(jnp.int32),
                     edge_index_cause[1].astype(jnp.int32), edge_weight_cause)

    p = _mm(xm, W_rel_corr_0)
    r_a = _mm(xa, W_root_cause_0)

    s = _seg_reduce(p, ec, n_tables=1, is_max=False)[:N_NODES]
    xm_blk = jnp.concatenate([xm[:, :128], xm[:, 128:]], axis=0)
    m0f = _seg_reduce(xm_blk, ea, n_tables=2, is_max=True)
    m0 = jnp.concatenate([m0f[:N_NODES], m0f[ROWS_PAD:ROWS_PAD + N_NODES]],
                         axis=1)

    xm1 = _fused(s, xm, W_root_corr_0, b_rel_corr_0)
    xa1 = _fused(r_a, m0, W_rel_cause_0, b_rel_cause_0)

    m1 = _seg_reduce(xm1, ea, n_tables=1, is_max=True)[:N_NODES]
    t = _mm(m1, W_rel_cause_1)
    xa2 = _fused(t, xa1, W_root_cause_1, b_rel_cause_1)
    return xa2
